# baseline (speedup 1.0000x reference)
import jax
import jax.numpy as jnp
from jax import lax
from jax.experimental import pallas as pl
from jax.experimental.pallas import tpu as pltpu
from jax.experimental.pallas import tpu_sc as plsc

_NC = 2
_NS = 16
_NW = _NC * _NS

_BATCH = 16384
_HIST = 200
_D = 32
_BPW = _BATCH // _NW
_L = 16
_NB = _HIST // 8
_TPB = _BPW // 128
_OPAD = 521


def _gather_body(tokens_hbm, table_hbm, out_hbm,
                 tokblk, idx8, rows0, rows1, outt0, outt1,
                 gsem0, gsem1, osem0, osem1):
    wid = lax.axis_index("s") * _NC + lax.axis_index("c")
    b0 = pl.multiple_of(wid * _BPW, _BPW)
    bt0 = pl.multiple_of(wid * _TPB, _TPB)
    lanes = lax.iota(jnp.int32, _L)
    lanes_hi = lanes + _L
    rows_ = (rows0, rows1)
    outt_ = (outt0, outt1)
    gsem_ = (gsem0, gsem1)
    osem_ = (osem0, osem1)

    def extract_band(band):
        pltpu.sync_copy(tokens_hbm.at[band, pl.ds(bt0, _TPB)], tokblk)
        for r in range(8):
            for g in range(_BPW // _L):
                idx8[r, pl.ds(g * _L, _L)] = (
                    tokblk[g // 8, r, pl.ds((g % 8) * _L, _L)]
                )

    def fire(r):
        pltpu.async_copy(table_hbm.at[idx8.at[r]], rows_[r % 2],
                         gsem_[r % 2])

    def transpose(rows, outt):
        @plsc.parallel_loop(0, _BPW // _L, unroll=4)
        def g_body(g):
            for j in range(_L):
                b = g * _L + j
                col = jnp.full((_L,), b, jnp.int32)
                v0 = rows[b, pl.ds(0, _L)]
                v1 = rows[b, pl.ds(_L, _L)]
                plsc.store_scatter(outt, [lanes, col], v0)
                plsc.store_scatter(outt, [lanes_hi, col], v1)

    def band_body(band, carry):
        extract_band(band)
        fire(0)
        for r in range(8):
            h = band * 8 + r
            par = r % 2
            pltpu.make_async_copy(table_hbm.at[idx8.at[r]], rows_[par],
                                  gsem_[par]).wait()

            @pl.when(h >= 2)
            def _():
                pltpu.make_async_copy(
                    outt_[par].at[:, pl.ds(0, _BPW)],
                    out_hbm.at[h - 2, :, pl.ds(b0, _BPW)], osem_[par]).wait()

            if r < 7:
                fire(r + 1)
            transpose(rows_[par], outt_[par])
            pltpu.async_copy(outt_[par].at[:, pl.ds(0, _BPW)],
                             out_hbm.at[h, :, pl.ds(b0, _BPW)], osem_[par])
        return carry

    lax.fori_loop(0, _NB, band_body, 0)
    pltpu.make_async_copy(outt0.at[:, pl.ds(0, _BPW)],
                          out_hbm.at[_HIST - 2, :, pl.ds(b0, _BPW)],
                          osem0).wait()
    pltpu.make_async_copy(outt1.at[:, pl.ds(0, _BPW)],
                          out_hbm.at[_HIST - 1, :, pl.ds(b0, _BPW)],
                          osem1).wait()


_sc_gather = pl.kernel(
    _gather_body,
    out_type=jax.ShapeDtypeStruct((_HIST, _D, _BATCH), jnp.float32),
    mesh=plsc.VectorSubcoreMesh(core_axis_name="c", subcore_axis_name="s"),
    scratch_types=[
        pltpu.VMEM((_TPB, 8, 128), jnp.int32),
        pltpu.VMEM((8, _BPW), jnp.int32),
        pltpu.VMEM((_BPW, _D), jnp.float32),
        pltpu.VMEM((_BPW, _D), jnp.float32),
        pltpu.VMEM((_D, _OPAD), jnp.float32),
        pltpu.VMEM((_D, _OPAD), jnp.float32),
        pltpu.SemaphoreType.DMA,
        pltpu.SemaphoreType.DMA,
        pltpu.SemaphoreType.DMA,
        pltpu.SemaphoreType.DMA,
    ],
    compiler_params=pltpu.CompilerParams(
        use_tc_tiling_on_sc=False, needs_layout_passes=False
    ),
)


@jax.jit
def kernel(tokens, embedding_weights):
    tok_tiles = (
        tokens.astype(jnp.int32).T
        .reshape(_NB, 8, 128, 128)
        .transpose(0, 2, 1, 3)
    )
    out_heb = _sc_gather(tok_tiles, embedding_weights)
    return jnp.transpose(out_heb, (2, 0, 1))


# speedup vs baseline: 1.2858x; 1.2858x over previous
import jax
import jax.numpy as jnp
from jax import lax
from jax.experimental import pallas as pl
from jax.experimental.pallas import tpu as pltpu
from jax.experimental.pallas import tpu_sc as plsc

_NC = 2
_NS = 16
_NW = _NC * _NS

_BATCH = 16384
_HIST = 200
_D = 32
_BPW = _BATCH // _NW
_L = 16
_NB = _HIST // 8
_TPB = _BPW // 128
_EB = _D // 8
_OPAD = 521


def _gather_body(tokens_hbm, table_hbm, out_hbm,
                 tokblk, idx8, rows0, rows1, outt0, outt1,
                 gsem0, gsem1, osem0, osem1):
    wid = lax.axis_index("s") * _NC + lax.axis_index("c")
    bt0 = pl.multiple_of(wid * _TPB, _TPB)
    lanes = lax.iota(jnp.int32, _L)
    rows_ = (rows0, rows1)
    outt_ = (outt0, outt1)
    gsem_ = (gsem0, gsem1)
    osem_ = (osem0, osem1)

    def extract_band(band):
        pltpu.sync_copy(tokens_hbm.at[band, pl.ds(bt0, _TPB)], tokblk)
        for r in range(8):
            for g in range(_BPW // _L):
                idx8[r, pl.ds(g * _L, _L)] = (
                    tokblk[g // 8, r, pl.ds((g % 8) * _L, _L)]
                )

    def fire(r):
        pltpu.async_copy(table_hbm.at[idx8.at[r]], rows_[r % 2],
                         gsem_[r % 2])

    lanes_hi = lanes + _L

    def transpose(rows, outt):
        @plsc.parallel_loop(0, _BPW // _L, unroll=2)
        def g_body(g):
            for j in range(_L):
                b = g * _L + j
                col = jnp.full((_L,), b, jnp.int32)
                v0 = rows[b, pl.ds(0, _L)]
                v1 = rows[b, pl.ds(_L, _L)]
                plsc.store_scatter(outt, [lanes, col], v0)
                plsc.store_scatter(outt, [lanes_hi, col], v1)

    def out_copies(h, par, start):
        for eb in range(_EB):
            for bt in range(_TPB):
                cp = pltpu.make_async_copy(
                    outt_[par].at[pl.ds(eb * 8, 8), pl.ds(bt * 128, 128)],
                    out_hbm.at[h, eb, bt0 + bt], osem_[par])
                if start:
                    cp.start()
                else:
                    cp.wait()

    def band_body(band, carry):
        extract_band(band)
        fire(0)
        for r in range(8):
            h = band * 8 + r
            par = r % 2
            pltpu.make_async_copy(table_hbm.at[idx8.at[r]], rows_[par],
                                  gsem_[par]).wait()

            @pl.when(h >= 2)
            def _():
                out_copies(h - 2, par, start=False)

            if r < 7:
                fire(r + 1)
            transpose(rows_[par], outt_[par])
            out_copies(h, par, start=True)
        return carry

    lax.fori_loop(0, _NB, band_body, 0)
    out_copies(_HIST - 2, 0, start=False)
    out_copies(_HIST - 1, 1, start=False)


_sc_gather = pl.kernel(
    _gather_body,
    out_type=jax.ShapeDtypeStruct((_HIST, _EB, 128, 8, 128), jnp.float32),
    mesh=plsc.VectorSubcoreMesh(core_axis_name="c", subcore_axis_name="s"),
    scratch_types=[
        pltpu.VMEM((_TPB, 8, 128), jnp.int32),
        pltpu.VMEM((8, _BPW), jnp.int32),
        pltpu.VMEM((_BPW, _D), jnp.float32),
        pltpu.VMEM((_BPW, _D), jnp.float32),
        pltpu.VMEM((_D, _OPAD), jnp.float32),
        pltpu.VMEM((_D, _OPAD), jnp.float32),
        pltpu.SemaphoreType.DMA,
        pltpu.SemaphoreType.DMA,
        pltpu.SemaphoreType.DMA,
        pltpu.SemaphoreType.DMA,
    ],
    compiler_params=pltpu.CompilerParams(
        use_tc_tiling_on_sc=False, needs_layout_passes=False
    ),
)


@jax.jit
def kernel(tokens, embedding_weights):
    tok_tiles = (
        tokens.astype(jnp.int32).T
        .reshape(_NB, 8, 128, 128)
        .transpose(0, 2, 1, 3)
    )
    out5 = _sc_gather(tok_tiles, embedding_weights)
    out = out5.transpose(2, 4, 0, 1, 3).reshape(_BATCH, _HIST, _D)
    return out
